# Initial kernel scaffold; baseline (speedup 1.0000x reference)
import functools

import jax
import jax.numpy as jnp
from jax.experimental import pallas as pl
from jax.experimental.pallas import tpu as pltpu

_I32_MIN = jnp.int32(-(2**31))
_I32_MAX = jnp.int32(2**31 - 1)


def _matvec_kernel(x_ref, w_ref, b_ref, out_ref):
    x = x_ref[0]
    res = jnp.dot(x, w_ref[...], preferred_element_type=jnp.float32) + b_ref[0]
    out_ref[...] = res[None]


def _avg_i32(lo, hi):
    return (lo >> 1) + (hi >> 1) + (lo & hi & jnp.int32(1))


def _select_kernel(s_ref, lab_ref, mask_ref, kv_ref, kb_ref, *, n_valid, k_top, k_bot):
    R, L = s_ref.shape
    s = s_ref[...]
    ib = jax.lax.bitcast_convert_type(s, jnp.int32)
    keys = jnp.where(ib >= 0, ib, ib ^ jnp.int32(0x7FFFFFFF))
    row = jax.lax.broadcasted_iota(jnp.int32, (R, L), 0)
    col = jax.lax.broadcasted_iota(jnp.int32, (R, L), 1)
    idx = row * L + col
    valid = idx < n_valid
    kv_ref[...] = jnp.where(valid, keys, _I32_MIN)
    kb_ref[...] = jnp.where(valid, keys, _I32_MAX)

    kt = jnp.int32(k_top)
    kb_k = jnp.int32(k_bot)

    def top_body(_, carry):
        lo, hi = carry
        mid = _avg_i32(lo, hi)
        c = jnp.sum((kv_ref[...] >= mid).astype(jnp.int32))
        take = c >= kt
        return jnp.where(take, mid, lo), jnp.where(take, hi, mid)

    vstar, _ = jax.lax.fori_loop(0, 32, top_body, (_I32_MIN, _I32_MAX))
    c_gt = jnp.sum((kv_ref[...] > vstar).astype(jnp.int32))
    need_top = kt - c_gt

    def ttop_body(_, carry):
        lo, hi = carry
        mid = (lo + hi) // 2
        c = jnp.sum(((kv_ref[...] == vstar) & (idx < mid)).astype(jnp.int32))
        take = c >= need_top
        return jnp.where(take, lo, mid), jnp.where(take, mid, hi)

    _, t_top = jax.lax.fori_loop(
        0, 17, ttop_body, (jnp.int32(0), jnp.int32(R * L))
    )

    def bot_body(_, carry):
        lo, hi = carry
        mid = _avg_i32(lo, hi)
        c = jnp.sum((kb_ref[...] <= mid).astype(jnp.int32))
        take = c >= kb_k
        return jnp.where(take, lo, mid), jnp.where(take, mid, hi)

    _, wstar = jax.lax.fori_loop(0, 32, bot_body, (_I32_MIN, _I32_MAX))
    c_lt = jnp.sum((kb_ref[...] < wstar).astype(jnp.int32))
    need_bot = kb_k - c_lt

    def tbot_body(_, carry):
        lo, hi = carry
        mid = (lo + hi) // 2
        c = jnp.sum(((kb_ref[...] == wstar) & (idx < mid)).astype(jnp.int32))
        take = c >= need_bot
        return jnp.where(take, lo, mid), jnp.where(take, mid, hi)

    _, t_bot = jax.lax.fori_loop(
        0, 17, tbot_body, (jnp.int32(0), jnp.int32(R * L))
    )

    kv = kv_ref[...]
    kb = kb_ref[...]
    top_sel = (kv > vstar) | ((kv == vstar) & (idx < t_top))
    bot_sel = (kb < wstar) | ((kb == wstar) & (idx < t_bot))
    one = jnp.float32(1.0)
    zero = jnp.float32(0.0)
    lab_ref[...] = jnp.where(top_sel & jnp.logical_not(bot_sel), one, zero)
    mask_ref[...] = jnp.where(top_sel | bot_sel, one, zero)


def _build(N, D, interpret=False):
    k_top = int(0.1 * N)
    k_bot = int(0.2 * N)
    NB = 25
    BN = N // NB
    L = 128
    R = (N + L - 1) // L
    if R % 8:
        R += 8 - R % 8

    matvec = pl.pallas_call(
        _matvec_kernel,
        grid=(NB,),
        in_specs=[
            pl.BlockSpec((1, BN, D), lambda i: (i, 0, 0)),
            pl.BlockSpec((D, 1), lambda i: (0, 0)),
            pl.BlockSpec(memory_space=pltpu.SMEM),
        ],
        out_specs=pl.BlockSpec((1, BN, 1), lambda i: (i, 0, 0)),
        out_shape=jax.ShapeDtypeStruct((NB, BN, 1), jnp.float32),
        interpret=interpret,
    )

    select = pl.pallas_call(
        functools.partial(_select_kernel, n_valid=N, k_top=k_top, k_bot=k_bot),
        in_specs=[pl.BlockSpec((R, L), lambda: (0, 0))],
        out_specs=[
            pl.BlockSpec((R, L), lambda: (0, 0)),
            pl.BlockSpec((R, L), lambda: (0, 0)),
        ],
        out_shape=[
            jax.ShapeDtypeStruct((R, L), jnp.float32),
            jax.ShapeDtypeStruct((R, L), jnp.float32),
        ],
        scratch_shapes=[
            pltpu.VMEM((R, L), jnp.int32),
            pltpu.VMEM((R, L), jnp.int32),
        ],
        interpret=interpret,
    )
    return matvec, select, R, L


def kernel(instances, bag_label, W, b, interpret=False):
    _, N, D = instances.shape
    matvec, select, R, L = _build(N, D, interpret=interpret)
    x3 = instances.reshape(-1, N // 25, D)
    preds = matvec(x3, W, b)
    scores = preds.reshape(N)
    s2d = jnp.pad(scores, (0, R * L - N)).reshape(R, L)
    lab2d, mask2d = select(s2d)
    preds_out = preds.reshape(1, N, 1)
    labels = lab2d.reshape(-1)[:N].reshape(1, N, 1)
    mask = mask2d.reshape(-1)[:N].reshape(1, N, 1)
    neg = bag_label[0] == 0
    labels = jnp.where(neg, jnp.zeros_like(labels), labels)
    mask = jnp.where(neg, jnp.ones_like(mask), mask)
    return preds_out, labels, mask


# speedup vs baseline: 3.2500x; 3.2500x over previous
import functools

import jax
import jax.numpy as jnp
from jax.experimental import pallas as pl
from jax.experimental.pallas import tpu as pltpu

_I32_MIN = -(2**31)
_I32_MAX = 2**31 - 1


def _matvec_kernel(x_ref, w_ref, b_ref, out_ref):
    x = x_ref[0]
    res = jnp.dot(x, w_ref[...], preferred_element_type=jnp.float32) + b_ref[0]
    out_ref[...] = res[None]


def _avg_i32(lo, hi):
    return (lo >> 1) + (hi >> 1) + (lo & hi & jnp.int32(1))


def _select_kernel(s_ref, lab_ref, mask_ref, kv_ref, kb_ref, *, n_valid, k_top, k_bot):
    R, L = s_ref.shape
    s = s_ref[...]
    ib = jax.lax.bitcast_convert_type(s, jnp.int32)
    keys = jnp.where(ib >= 0, ib, ib ^ jnp.int32(0x7FFFFFFF))
    row = jax.lax.broadcasted_iota(jnp.int32, (R, L), 0)
    col = jax.lax.broadcasted_iota(jnp.int32, (R, L), 1)
    idx = row * L + col
    valid = idx < n_valid
    imin = jnp.int32(_I32_MIN)
    imax = jnp.int32(_I32_MAX)
    kv_ref[...] = jnp.where(valid, keys, imin)
    kb_ref[...] = jnp.where(valid, keys, imax)

    kt = jnp.int32(k_top)
    kb_k = jnp.int32(k_bot)

    def top_body(_, carry):
        lo, hi = carry
        mid = _avg_i32(lo, hi)
        c = jnp.sum((kv_ref[...] >= mid).astype(jnp.int32))
        take = c >= kt
        return jnp.where(take, mid, lo), jnp.where(take, hi, mid)

    vstar, _ = jax.lax.fori_loop(0, 32, top_body, (imin, imax))
    c_gt = jnp.sum((kv_ref[...] > vstar).astype(jnp.int32))
    need_top = kt - c_gt

    def ttop_body(_, carry):
        lo, hi = carry
        mid = (lo + hi) // 2
        c = jnp.sum(((kv_ref[...] == vstar) & (idx < mid)).astype(jnp.int32))
        take = c >= need_top
        return jnp.where(take, lo, mid), jnp.where(take, mid, hi)

    _, t_top = jax.lax.fori_loop(
        0, 17, ttop_body, (jnp.int32(0), jnp.int32(R * L))
    )

    def bot_body(_, carry):
        lo, hi = carry
        mid = _avg_i32(lo, hi)
        c = jnp.sum((kb_ref[...] <= mid).astype(jnp.int32))
        take = c >= kb_k
        return jnp.where(take, lo, mid), jnp.where(take, mid, hi)

    _, wstar = jax.lax.fori_loop(0, 32, bot_body, (imin, imax))
    c_lt = jnp.sum((kb_ref[...] < wstar).astype(jnp.int32))
    need_bot = kb_k - c_lt

    def tbot_body(_, carry):
        lo, hi = carry
        mid = (lo + hi) // 2
        c = jnp.sum(((kb_ref[...] == wstar) & (idx < mid)).astype(jnp.int32))
        take = c >= need_bot
        return jnp.where(take, lo, mid), jnp.where(take, mid, hi)

    _, t_bot = jax.lax.fori_loop(
        0, 17, tbot_body, (jnp.int32(0), jnp.int32(R * L))
    )

    kv = kv_ref[...]
    kb = kb_ref[...]
    top_sel = (kv > vstar) | ((kv == vstar) & (idx < t_top))
    bot_sel = (kb < wstar) | ((kb == wstar) & (idx < t_bot))
    one = jnp.float32(1.0)
    zero = jnp.float32(0.0)
    lab_ref[...] = jnp.where(top_sel & jnp.logical_not(bot_sel), one, zero)
    mask_ref[...] = jnp.where(top_sel | bot_sel, one, zero)


def _build(N, D, interpret=False):
    k_top = int(0.1 * N)
    k_bot = int(0.2 * N)
    NB = 25
    BN = N // NB
    L = 128
    R = (N + L - 1) // L
    if R % 8:
        R += 8 - R % 8

    matvec = pl.pallas_call(
        _matvec_kernel,
        grid=(NB,),
        in_specs=[
            pl.BlockSpec((1, BN, D), lambda i: (i, 0, 0)),
            pl.BlockSpec((D, 1), lambda i: (0, 0)),
            pl.BlockSpec(memory_space=pltpu.SMEM),
        ],
        out_specs=pl.BlockSpec((1, BN, 1), lambda i: (i, 0, 0)),
        out_shape=jax.ShapeDtypeStruct((NB, BN, 1), jnp.float32),
        interpret=interpret,
    )

    select = pl.pallas_call(
        functools.partial(_select_kernel, n_valid=N, k_top=k_top, k_bot=k_bot),
        in_specs=[pl.BlockSpec((R, L), lambda: (0, 0))],
        out_specs=[
            pl.BlockSpec((R, L), lambda: (0, 0)),
            pl.BlockSpec((R, L), lambda: (0, 0)),
        ],
        out_shape=[
            jax.ShapeDtypeStruct((R, L), jnp.float32),
            jax.ShapeDtypeStruct((R, L), jnp.float32),
        ],
        scratch_shapes=[
            pltpu.VMEM((R, L), jnp.int32),
            pltpu.VMEM((R, L), jnp.int32),
        ],
        interpret=interpret,
    )
    return matvec, select, R, L


def kernel(instances, bag_label, W, b, interpret=False):
    _, N, D = instances.shape
    matvec, select, R, L = _build(N, D, interpret=interpret)
    x3 = instances.reshape(-1, N // 25, D)
    preds = matvec(x3, W, b)
    scores = preds.reshape(N)
    s2d = jnp.pad(scores, (0, R * L - N)).reshape(R, L)
    lab2d, mask2d = select(s2d)
    preds_out = preds.reshape(1, N, 1)
    labels = lab2d.reshape(-1)[:N].reshape(1, N, 1)
    mask = mask2d.reshape(-1)[:N].reshape(1, N, 1)
    neg = bag_label[0] == 0
    labels = jnp.where(neg, jnp.zeros_like(labels), labels)
    mask = jnp.where(neg, jnp.ones_like(mask), mask)
    return preds_out, labels, mask


# speedup vs baseline: 3.8441x; 1.1828x over previous
import functools

import jax
import jax.numpy as jnp
from jax.experimental import pallas as pl
from jax.experimental.pallas import tpu as pltpu

_I32_MIN = -(2**31)
_I32_MAX = 2**31 - 1


def _matvec_kernel(x_ref, w_ref, b_ref, out_ref):
    x = x_ref[0]
    res = jnp.dot(x, w_ref[...], preferred_element_type=jnp.float32) + b_ref[0]
    out_ref[...] = res[None]


def _avg_i32(lo, hi):
    return (lo >> 1) + (hi >> 1) + (lo & hi & jnp.int32(1))


def _select_kernel(s_ref, lab_ref, mask_ref, kv_ref, kb_ref, *, n_valid, k_top, k_bot):
    R, L = s_ref.shape
    s = s_ref[...]
    ib = jax.lax.bitcast_convert_type(s, jnp.int32)
    keys = jnp.where(ib >= 0, ib, ib ^ jnp.int32(0x7FFFFFFF))
    row = jax.lax.broadcasted_iota(jnp.int32, (R, L), 0)
    col = jax.lax.broadcasted_iota(jnp.int32, (R, L), 1)
    idx = row * L + col
    valid = idx < n_valid
    imin = jnp.int32(_I32_MIN)
    imax = jnp.int32(_I32_MAX)
    kv_ref[...] = jnp.where(valid, keys, imin)
    kb_ref[...] = jnp.where(valid, keys, imax)

    kt = jnp.int32(k_top)
    kb_k = jnp.int32(k_bot)

    def val_body(_, carry):
        lo_t, hi_t, lo_b, hi_b = carry
        mid_t = _avg_i32(lo_t, hi_t)
        mid_b = _avg_i32(lo_b, hi_b)
        kv = kv_ref[...]
        kb = kb_ref[...]
        c_t = jnp.sum((kv >= mid_t).astype(jnp.int32))
        c_b = jnp.sum((kb <= mid_b).astype(jnp.int32))
        take_t = c_t >= kt
        take_b = c_b >= kb_k
        return (
            jnp.where(take_t, mid_t, lo_t),
            jnp.where(take_t, hi_t, mid_t),
            jnp.where(take_b, lo_b, mid_b),
            jnp.where(take_b, mid_b, hi_b),
        )

    vstar, _, _, wstar = jax.lax.fori_loop(
        0, 32, val_body, (imin, imax, imin, imax)
    )

    kv = kv_ref[...]
    kb = kb_ref[...]
    c_gt = jnp.sum((kv > vstar).astype(jnp.int32))
    eq_t = jnp.sum((kv == vstar).astype(jnp.int32))
    c_lt = jnp.sum((kb < wstar).astype(jnp.int32))
    eq_b = jnp.sum((kb == wstar).astype(jnp.int32))
    need_top = kt - c_gt
    need_bot = kb_k - c_lt
    rl = jnp.int32(R * L)

    def ttop_search():
        def body(_, carry):
            lo, hi = carry
            mid = (lo + hi) // 2
            c = jnp.sum(((kv_ref[...] == vstar) & (idx < mid)).astype(jnp.int32))
            take = c >= need_top
            return jnp.where(take, lo, mid), jnp.where(take, mid, hi)

        _, t = jax.lax.fori_loop(0, 17, body, (jnp.int32(0), rl))
        return t

    t_top = jax.lax.cond(eq_t == need_top, lambda: rl, ttop_search)

    def tbot_search():
        def body(_, carry):
            lo, hi = carry
            mid = (lo + hi) // 2
            c = jnp.sum(((kb_ref[...] == wstar) & (idx < mid)).astype(jnp.int32))
            take = c >= need_bot
            return jnp.where(take, lo, mid), jnp.where(take, mid, hi)

        _, t = jax.lax.fori_loop(0, 17, body, (jnp.int32(0), rl))
        return t

    t_bot = jax.lax.cond(eq_b == need_bot, lambda: rl, tbot_search)

    kv = kv_ref[...]
    kb = kb_ref[...]
    top_sel = (kv > vstar) | ((kv == vstar) & (idx < t_top))
    bot_sel = (kb < wstar) | ((kb == wstar) & (idx < t_bot))
    one = jnp.float32(1.0)
    zero = jnp.float32(0.0)
    lab_ref[...] = jnp.where(top_sel & jnp.logical_not(bot_sel), one, zero)
    mask_ref[...] = jnp.where(top_sel | bot_sel, one, zero)


def _build(N, D, interpret=False):
    k_top = int(0.1 * N)
    k_bot = int(0.2 * N)
    NB = 25
    BN = N // NB
    L = 128
    R = (N + L - 1) // L
    if R % 8:
        R += 8 - R % 8

    matvec = pl.pallas_call(
        _matvec_kernel,
        grid=(NB,),
        in_specs=[
            pl.BlockSpec((1, BN, D), lambda i: (i, 0, 0)),
            pl.BlockSpec((D, 1), lambda i: (0, 0)),
            pl.BlockSpec(memory_space=pltpu.SMEM),
        ],
        out_specs=pl.BlockSpec((1, BN, 1), lambda i: (i, 0, 0)),
        out_shape=jax.ShapeDtypeStruct((NB, BN, 1), jnp.float32),
        interpret=interpret,
    )

    select = pl.pallas_call(
        functools.partial(_select_kernel, n_valid=N, k_top=k_top, k_bot=k_bot),
        in_specs=[pl.BlockSpec((R, L), lambda: (0, 0))],
        out_specs=[
            pl.BlockSpec((R, L), lambda: (0, 0)),
            pl.BlockSpec((R, L), lambda: (0, 0)),
        ],
        out_shape=[
            jax.ShapeDtypeStruct((R, L), jnp.float32),
            jax.ShapeDtypeStruct((R, L), jnp.float32),
        ],
        scratch_shapes=[
            pltpu.VMEM((R, L), jnp.int32),
            pltpu.VMEM((R, L), jnp.int32),
        ],
        interpret=interpret,
    )
    return matvec, select, R, L


def kernel(instances, bag_label, W, b, interpret=False):
    _, N, D = instances.shape
    matvec, select, R, L = _build(N, D, interpret=interpret)
    x3 = instances.reshape(-1, N // 25, D)
    preds = matvec(x3, W, b)
    scores = preds.reshape(N)
    s2d = jnp.pad(scores, (0, R * L - N)).reshape(R, L)
    lab2d, mask2d = select(s2d)
    preds_out = preds.reshape(1, N, 1)
    labels = lab2d.reshape(-1)[:N].reshape(1, N, 1)
    mask = mask2d.reshape(-1)[:N].reshape(1, N, 1)
    neg = bag_label[0] == 0
    labels = jnp.where(neg, jnp.zeros_like(labels), labels)
    mask = jnp.where(neg, jnp.ones_like(mask), mask)
    return preds_out, labels, mask


# speedup vs baseline: 5.5184x; 1.4355x over previous
import functools

import jax
import jax.numpy as jnp
from jax.experimental import pallas as pl
from jax.experimental.pallas import tpu as pltpu

_I32_MIN = -(2**31)
_I32_MAX = 2**31 - 1


def _matvec_kernel(x_ref, w_ref, b_ref, out_ref):
    x = x_ref[0]
    res = jnp.dot(x, w_ref[...], preferred_element_type=jnp.float32) + b_ref[0]
    out_ref[...] = res.reshape(out_ref.shape)


def _avg_i32(lo, hi):
    return (lo >> 1) + (hi >> 1) + (lo & hi & jnp.int32(1))


def _select_kernel(s_ref, lab_ref, mask_ref, kv_ref, kb_ref, *, n_valid, k_top, k_bot):
    R, L = s_ref.shape
    s = s_ref[...]
    ib = jax.lax.bitcast_convert_type(s, jnp.int32)
    keys = jnp.where(ib >= 0, ib, ib ^ jnp.int32(0x7FFFFFFF))
    row = jax.lax.broadcasted_iota(jnp.int32, (R, L), 0)
    col = jax.lax.broadcasted_iota(jnp.int32, (R, L), 1)
    idx = row * L + col
    valid = idx < n_valid
    imin = jnp.int32(_I32_MIN)
    imax = jnp.int32(_I32_MAX)
    kv_ref[...] = jnp.where(valid, keys, imin)
    kb_ref[...] = jnp.where(valid, keys, imax)

    kt = jnp.int32(k_top)
    kb_k = jnp.int32(k_bot)

    def val_body(_, carry):
        lo_t, hi_t, lo_b, hi_b = carry
        mid_t = _avg_i32(lo_t, hi_t)
        mid_b = _avg_i32(lo_b, hi_b)
        kv = kv_ref[...]
        kb = kb_ref[...]
        c_t = jnp.sum((kv >= mid_t).astype(jnp.int32))
        c_b = jnp.sum((kb <= mid_b).astype(jnp.int32))
        take_t = c_t >= kt
        take_b = c_b >= kb_k
        return (
            jnp.where(take_t, mid_t, lo_t),
            jnp.where(take_t, hi_t, mid_t),
            jnp.where(take_b, lo_b, mid_b),
            jnp.where(take_b, mid_b, hi_b),
        )

    vstar, _, _, wstar = jax.lax.fori_loop(
        0, 32, val_body, (imin, imax, imin, imax)
    )

    kv = kv_ref[...]
    kb = kb_ref[...]
    c_gt = jnp.sum((kv > vstar).astype(jnp.int32))
    eq_t = jnp.sum((kv == vstar).astype(jnp.int32))
    c_lt = jnp.sum((kb < wstar).astype(jnp.int32))
    eq_b = jnp.sum((kb == wstar).astype(jnp.int32))
    need_top = kt - c_gt
    need_bot = kb_k - c_lt
    rl = jnp.int32(R * L)

    def ttop_search():
        def body(_, carry):
            lo, hi = carry
            mid = (lo + hi) // 2
            c = jnp.sum(((kv_ref[...] == vstar) & (idx < mid)).astype(jnp.int32))
            take = c >= need_top
            return jnp.where(take, lo, mid), jnp.where(take, mid, hi)

        _, t = jax.lax.fori_loop(0, 17, body, (jnp.int32(0), rl))
        return t

    t_top = jax.lax.cond(eq_t == need_top, lambda: rl, ttop_search)

    def tbot_search():
        def body(_, carry):
            lo, hi = carry
            mid = (lo + hi) // 2
            c = jnp.sum(((kb_ref[...] == wstar) & (idx < mid)).astype(jnp.int32))
            take = c >= need_bot
            return jnp.where(take, lo, mid), jnp.where(take, mid, hi)

        _, t = jax.lax.fori_loop(0, 17, body, (jnp.int32(0), rl))
        return t

    t_bot = jax.lax.cond(eq_b == need_bot, lambda: rl, tbot_search)

    kv = kv_ref[...]
    kb = kb_ref[...]
    top_sel = (kv > vstar) | ((kv == vstar) & (idx < t_top))
    bot_sel = (kb < wstar) | ((kb == wstar) & (idx < t_bot))
    one = jnp.float32(1.0)
    zero = jnp.float32(0.0)
    lab_ref[...] = jnp.where(top_sel & jnp.logical_not(bot_sel), one, zero)
    mask_ref[...] = jnp.where(top_sel | bot_sel, one, zero)


def _build(N, D, interpret=False):
    k_top = int(0.1 * N)
    k_bot = int(0.2 * N)
    L = 128
    BR = 16
    BN = BR * L
    NB = (N + BN - 1) // BN
    R = NB * BR

    matvec = pl.pallas_call(
        _matvec_kernel,
        grid=(NB,),
        in_specs=[
            pl.BlockSpec((1, BN, D), lambda i: (0, i, 0)),
            pl.BlockSpec((D, 1), lambda i: (0, 0)),
            pl.BlockSpec(memory_space=pltpu.SMEM),
        ],
        out_specs=pl.BlockSpec((BR, L), lambda i: (i, 0)),
        out_shape=jax.ShapeDtypeStruct((R, L), jnp.float32),
        interpret=interpret,
    )

    select = pl.pallas_call(
        functools.partial(_select_kernel, n_valid=N, k_top=k_top, k_bot=k_bot),
        in_specs=[pl.BlockSpec((R, L), lambda: (0, 0))],
        out_specs=[
            pl.BlockSpec((R, L), lambda: (0, 0)),
            pl.BlockSpec((R, L), lambda: (0, 0)),
        ],
        out_shape=[
            jax.ShapeDtypeStruct((R, L), jnp.float32),
            jax.ShapeDtypeStruct((R, L), jnp.float32),
        ],
        scratch_shapes=[
            pltpu.VMEM((R, L), jnp.int32),
            pltpu.VMEM((R, L), jnp.int32),
        ],
        interpret=interpret,
    )
    return matvec, select, R, L


def kernel(instances, bag_label, W, b, interpret=False):
    _, N, D = instances.shape
    matvec, select, R, L = _build(N, D, interpret=interpret)
    s2d = matvec(instances, W, b)
    lab2d, mask2d = select(s2d)
    preds_out = s2d.reshape(-1)[:N].reshape(1, N, 1)
    labels = lab2d.reshape(-1)[:N].reshape(1, N, 1)
    mask = mask2d.reshape(-1)[:N].reshape(1, N, 1)
    neg = bag_label[0] == 0
    labels = jnp.where(neg, jnp.zeros_like(labels), labels)
    mask = jnp.where(neg, jnp.ones_like(mask), mask)
    return preds_out, labels, mask


# speedup vs baseline: 6.5218x; 1.1818x over previous
import functools

import jax
import jax.numpy as jnp
from jax.experimental import pallas as pl
from jax.experimental.pallas import tpu as pltpu

_I32_MIN = -(2**31)
_I32_MAX = 2**31 - 1


def _matvec_kernel(x_ref, w_ref, b_ref, out_ref):
    x = x_ref[0]
    res = jnp.dot(x, w_ref[...], preferred_element_type=jnp.float32) + b_ref[0]
    out_ref[...] = res.reshape(out_ref.shape)


def _avg_i32(lo, hi):
    return (lo >> 1) + (hi >> 1) + (lo & hi & jnp.int32(1))


def _select_kernel(s_ref, lab_ref, mask_ref, kv_ref, kb_ref, *, n_valid, k_top, k_bot):
    R, L = s_ref.shape
    s = s_ref[...]
    ib = jax.lax.bitcast_convert_type(s, jnp.int32)
    keys = jnp.where(ib >= 0, ib, ib ^ jnp.int32(0x7FFFFFFF))
    row = jax.lax.broadcasted_iota(jnp.int32, (R, L), 0)
    col = jax.lax.broadcasted_iota(jnp.int32, (R, L), 1)
    idx = row * L + col
    valid = idx < n_valid
    imin = jnp.int32(_I32_MIN)
    imax = jnp.int32(_I32_MAX)
    kv_ref[...] = jnp.where(valid, keys, imin)
    kb_ref[...] = jnp.where(valid, keys, imax)

    kt = jnp.int32(k_top)
    kb_k = jnp.int32(k_bot)

    def val_body(_, carry):
        lo_t, hi_t, lo_b, hi_b = carry
        mid_t = _avg_i32(lo_t, hi_t)
        mid_b = _avg_i32(lo_b, hi_b)
        kv = kv_ref[...]
        kb = kb_ref[...]
        c_t = jnp.sum((kv >= mid_t).astype(jnp.int32))
        c_b = jnp.sum((kb <= mid_b).astype(jnp.int32))
        take_t = c_t >= kt
        take_b = c_b >= kb_k
        return (
            jnp.where(take_t, mid_t, lo_t),
            jnp.where(take_t, hi_t, mid_t),
            jnp.where(take_b, lo_b, mid_b),
            jnp.where(take_b, mid_b, hi_b),
        )

    vstar, _, _, wstar = jax.lax.fori_loop(
        0, 32, val_body, (imin, imax, imin, imax)
    )

    kv = kv_ref[...]
    kb = kb_ref[...]
    c_gt = jnp.sum((kv > vstar).astype(jnp.int32))
    eq_t = jnp.sum((kv == vstar).astype(jnp.int32))
    c_lt = jnp.sum((kb < wstar).astype(jnp.int32))
    eq_b = jnp.sum((kb == wstar).astype(jnp.int32))
    need_top = kt - c_gt
    need_bot = kb_k - c_lt
    rl = jnp.int32(R * L)

    def ttop_search():
        def body(_, carry):
            lo, hi = carry
            mid = (lo + hi) // 2
            c = jnp.sum(((kv_ref[...] == vstar) & (idx < mid)).astype(jnp.int32))
            take = c >= need_top
            return jnp.where(take, lo, mid), jnp.where(take, mid, hi)

        _, t = jax.lax.fori_loop(0, 17, body, (jnp.int32(0), rl))
        return t

    t_top = jax.lax.cond(eq_t == need_top, lambda: rl, ttop_search)

    def tbot_search():
        def body(_, carry):
            lo, hi = carry
            mid = (lo + hi) // 2
            c = jnp.sum(((kb_ref[...] == wstar) & (idx < mid)).astype(jnp.int32))
            take = c >= need_bot
            return jnp.where(take, lo, mid), jnp.where(take, mid, hi)

        _, t = jax.lax.fori_loop(0, 17, body, (jnp.int32(0), rl))
        return t

    t_bot = jax.lax.cond(eq_b == need_bot, lambda: rl, tbot_search)

    kv = kv_ref[...]
    kb = kb_ref[...]
    top_sel = (kv > vstar) | ((kv == vstar) & (idx < t_top))
    bot_sel = (kb < wstar) | ((kb == wstar) & (idx < t_bot))
    one = jnp.float32(1.0)
    zero = jnp.float32(0.0)
    lab_ref[...] = jnp.where(top_sel & jnp.logical_not(bot_sel), one, zero)
    mask_ref[...] = jnp.where(top_sel | bot_sel, one, zero)


def _build(N, D, interpret=False):
    k_top = int(0.1 * N)
    k_bot = int(0.2 * N)
    L = 128
    BR = 32
    BN = BR * L
    NB = (N + BN - 1) // BN
    R = NB * BR

    matvec = pl.pallas_call(
        _matvec_kernel,
        grid=(NB,),
        in_specs=[
            pl.BlockSpec((1, BN, D), lambda i: (0, i, 0)),
            pl.BlockSpec((D, 1), lambda i: (0, 0)),
            pl.BlockSpec(memory_space=pltpu.SMEM),
        ],
        out_specs=pl.BlockSpec((BR, L), lambda i: (i, 0)),
        out_shape=jax.ShapeDtypeStruct((R, L), jnp.float32),
        interpret=interpret,
    )

    select = pl.pallas_call(
        functools.partial(_select_kernel, n_valid=N, k_top=k_top, k_bot=k_bot),
        in_specs=[pl.BlockSpec((R, L), lambda: (0, 0))],
        out_specs=[
            pl.BlockSpec((R, L), lambda: (0, 0)),
            pl.BlockSpec((R, L), lambda: (0, 0)),
        ],
        out_shape=[
            jax.ShapeDtypeStruct((R, L), jnp.float32),
            jax.ShapeDtypeStruct((R, L), jnp.float32),
        ],
        scratch_shapes=[
            pltpu.VMEM((R, L), jnp.int32),
            pltpu.VMEM((R, L), jnp.int32),
        ],
        interpret=interpret,
    )
    return matvec, select, R, L


def kernel(instances, bag_label, W, b, interpret=False):
    _, N, D = instances.shape
    matvec, select, R, L = _build(N, D, interpret=interpret)
    s2d = matvec(instances, W, b)
    lab2d, mask2d = select(s2d)
    preds_out = s2d.reshape(-1)[:N].reshape(1, N, 1)
    labels = lab2d.reshape(-1)[:N].reshape(1, N, 1)
    mask = mask2d.reshape(-1)[:N].reshape(1, N, 1)
    neg = bag_label[0] == 0
    labels = jnp.where(neg, jnp.zeros_like(labels), labels)
    mask = jnp.where(neg, jnp.ones_like(mask), mask)
    return preds_out, labels, mask


# speedup vs baseline: 7.1479x; 1.0960x over previous
import functools

import jax
import jax.numpy as jnp
from jax.experimental import pallas as pl
from jax.experimental.pallas import tpu as pltpu

_I32_MIN = -(2**31)
_I32_MAX = 2**31 - 1


def _matvec_kernel(x_ref, w_ref, b_ref, out_ref):
    x = x_ref[0]
    res = jnp.dot(x, w_ref[...], preferred_element_type=jnp.float32) + b_ref[0]
    out_ref[...] = res.reshape(out_ref.shape)


def _avg_i32(lo, hi):
    return (lo >> 1) + (hi >> 1) + (lo & hi & jnp.int32(1))


def _select_kernel(s_ref, lab_ref, mask_ref, kv_ref, kb_ref, *, n_valid, k_top, k_bot):
    R, L = s_ref.shape
    s = s_ref[...]
    ib = jax.lax.bitcast_convert_type(s, jnp.int32)
    keys = jnp.where(ib >= 0, ib, ib ^ jnp.int32(0x7FFFFFFF))
    row = jax.lax.broadcasted_iota(jnp.int32, (R, L), 0)
    col = jax.lax.broadcasted_iota(jnp.int32, (R, L), 1)
    idx = row * L + col
    valid = idx < n_valid
    imin = jnp.int32(_I32_MIN)
    imax = jnp.int32(_I32_MAX)
    kv_ref[...] = jnp.where(valid, keys, imin)
    kb_ref[...] = jnp.where(valid, keys, imax)

    kt = jnp.int32(k_top)
    kb_k = jnp.int32(k_bot)

    def val_body(_, carry):
        lo_t, hi_t, lo_b, hi_b = carry
        mid_t = _avg_i32(lo_t, hi_t)
        mid_b = _avg_i32(lo_b, hi_b)
        kv = kv_ref[...]
        kb = kb_ref[...]
        c_t = jnp.sum((kv >= mid_t).astype(jnp.int32))
        c_b = jnp.sum((kb <= mid_b).astype(jnp.int32))
        take_t = c_t >= kt
        take_b = c_b >= kb_k
        return (
            jnp.where(take_t, mid_t, lo_t),
            jnp.where(take_t, hi_t, mid_t),
            jnp.where(take_b, lo_b, mid_b),
            jnp.where(take_b, mid_b, hi_b),
        )

    vstar, _, _, wstar = jax.lax.fori_loop(
        0, 32, val_body, (imin, imax, imin, imax)
    )

    kv = kv_ref[...]
    kb = kb_ref[...]
    c_gt = jnp.sum((kv > vstar).astype(jnp.int32))
    eq_t = jnp.sum((kv == vstar).astype(jnp.int32))
    c_lt = jnp.sum((kb < wstar).astype(jnp.int32))
    eq_b = jnp.sum((kb == wstar).astype(jnp.int32))
    need_top = kt - c_gt
    need_bot = kb_k - c_lt
    rl = jnp.int32(R * L)

    def ttop_search():
        def body(_, carry):
            lo, hi = carry
            mid = (lo + hi) // 2
            c = jnp.sum(((kv_ref[...] == vstar) & (idx < mid)).astype(jnp.int32))
            take = c >= need_top
            return jnp.where(take, lo, mid), jnp.where(take, mid, hi)

        _, t = jax.lax.fori_loop(0, 17, body, (jnp.int32(0), rl))
        return t

    t_top = jax.lax.cond(eq_t == need_top, lambda: rl, ttop_search)

    def tbot_search():
        def body(_, carry):
            lo, hi = carry
            mid = (lo + hi) // 2
            c = jnp.sum(((kb_ref[...] == wstar) & (idx < mid)).astype(jnp.int32))
            take = c >= need_bot
            return jnp.where(take, lo, mid), jnp.where(take, mid, hi)

        _, t = jax.lax.fori_loop(0, 17, body, (jnp.int32(0), rl))
        return t

    t_bot = jax.lax.cond(eq_b == need_bot, lambda: rl, tbot_search)

    kv = kv_ref[...]
    kb = kb_ref[...]
    top_sel = (kv > vstar) | ((kv == vstar) & (idx < t_top))
    bot_sel = (kb < wstar) | ((kb == wstar) & (idx < t_bot))
    one = jnp.float32(1.0)
    zero = jnp.float32(0.0)
    lab_ref[...] = jnp.where(top_sel & jnp.logical_not(bot_sel), one, zero)
    mask_ref[...] = jnp.where(top_sel | bot_sel, one, zero)


def _build(N, D, interpret=False):
    k_top = int(0.1 * N)
    k_bot = int(0.2 * N)
    L = 128
    BR = 64
    BN = BR * L
    NB = (N + BN - 1) // BN
    R = NB * BR

    matvec = pl.pallas_call(
        _matvec_kernel,
        grid=(NB,),
        in_specs=[
            pl.BlockSpec((1, BN, D), lambda i: (0, i, 0)),
            pl.BlockSpec((D, 1), lambda i: (0, 0)),
            pl.BlockSpec(memory_space=pltpu.SMEM),
        ],
        out_specs=pl.BlockSpec((BR, L), lambda i: (i, 0)),
        out_shape=jax.ShapeDtypeStruct((R, L), jnp.float32),
        interpret=interpret,
    )

    select = pl.pallas_call(
        functools.partial(_select_kernel, n_valid=N, k_top=k_top, k_bot=k_bot),
        in_specs=[pl.BlockSpec((R, L), lambda: (0, 0))],
        out_specs=[
            pl.BlockSpec((R, L), lambda: (0, 0)),
            pl.BlockSpec((R, L), lambda: (0, 0)),
        ],
        out_shape=[
            jax.ShapeDtypeStruct((R, L), jnp.float32),
            jax.ShapeDtypeStruct((R, L), jnp.float32),
        ],
        scratch_shapes=[
            pltpu.VMEM((R, L), jnp.int32),
            pltpu.VMEM((R, L), jnp.int32),
        ],
        interpret=interpret,
    )
    return matvec, select, R, L


def kernel(instances, bag_label, W, b, interpret=False):
    _, N, D = instances.shape
    matvec, select, R, L = _build(N, D, interpret=interpret)
    s2d = matvec(instances, W, b)
    lab2d, mask2d = select(s2d)
    preds_out = s2d.reshape(-1)[:N].reshape(1, N, 1)
    labels = lab2d.reshape(-1)[:N].reshape(1, N, 1)
    mask = mask2d.reshape(-1)[:N].reshape(1, N, 1)
    neg = bag_label[0] == 0
    labels = jnp.where(neg, jnp.zeros_like(labels), labels)
    mask = jnp.where(neg, jnp.ones_like(mask), mask)
    return preds_out, labels, mask
